# baseline (speedup 1.0000x reference)
import jax
import jax.numpy as jnp
from jax.experimental import pallas as pl
from jax.experimental.pallas import tpu as pltpu

I = 1024
S = 1024
O = 1024
G = 8
BR = (3 * S) // G
BO = O // G


def _fused_step(f_ref, h_ref, wih_ref, whh_ref, bih_ref, bhh_ref,
                wout_ref, bout_ref, out_logp_ref, out_h_ref,
                gi_ref, gh_ref, lg_ref):
    g = pl.program_id(0)

    f_row = f_ref[...]
    h_row = h_ref[...]

    gi = jnp.sum(wih_ref[...] * f_row, axis=1)
    gh = jnp.sum(whh_ref[...] * h_row, axis=1)
    gi_ref[0, pl.ds(g * BR, BR)] = gi
    gh_ref[0, pl.ds(g * BR, BR)] = gh

    pred = jnp.concatenate([f_row, h_row, h_row], axis=1)
    lo = jnp.sum(wout_ref[:, :3 * S] * pred, axis=1)
    lg_ref[0, pl.ds(g * BO, BO)] = lo

    @pl.when(g == G - 1)
    def _finish():
        gi_full = gi_ref[0, :] + bih_ref[0, :]
        gh_full = gh_ref[0, :] + bhh_ref[0, :]
        r = jax.nn.sigmoid(gi_full[:S] + gh_full[:S])
        z = jax.nn.sigmoid(gi_full[S:2 * S] + gh_full[S:2 * S])
        n = jnp.tanh(gi_full[2 * S:] + r * gh_full[2 * S:])
        h_vec = h_ref[0, :]
        out_h_ref[0, :] = (1.0 - z) * n + z * h_vec

        logits = lg_ref[0, :] + bout_ref[0, :]
        m = jnp.max(logits)
        lse = jnp.log(jnp.sum(jnp.exp(logits - m))) + m
        out_logp_ref[0, :] = logits - lse


def kernel(feature, time, initial_h, W_ih, W_hh, b_ih, b_hh, W_out, b_out):
    del time
    f_row = feature.reshape(1, I)
    h_row = initial_h.reshape(1, S)
    bih_row = b_ih.reshape(1, 3 * S)
    bhh_row = b_hh.reshape(1, 3 * S)
    bout_row = b_out.reshape(1, O)

    out_logp, out_h = pl.pallas_call(
        _fused_step,
        grid=(G,),
        in_specs=[
            pl.BlockSpec((1, I), lambda g: (0, 0)),
            pl.BlockSpec((1, S), lambda g: (0, 0)),
            pl.BlockSpec((BR, I), lambda g: (g, 0)),
            pl.BlockSpec((BR, S), lambda g: (g, 0)),
            pl.BlockSpec((1, 3 * S), lambda g: (0, 0)),
            pl.BlockSpec((1, 3 * S), lambda g: (0, 0)),
            pl.BlockSpec((BO, W_out.shape[1]), lambda g: (g, 0)),
            pl.BlockSpec((1, O), lambda g: (0, 0)),
        ],
        out_specs=[
            pl.BlockSpec((1, O), lambda g: (0, 0)),
            pl.BlockSpec((1, S), lambda g: (0, 0)),
        ],
        out_shape=[
            jax.ShapeDtypeStruct((1, O), jnp.float32),
            jax.ShapeDtypeStruct((1, S), jnp.float32),
        ],
        scratch_shapes=[
            pltpu.VMEM((1, 3 * S), jnp.float32),
            pltpu.VMEM((1, 3 * S), jnp.float32),
            pltpu.VMEM((1, O), jnp.float32),
        ],
    )(f_row, h_row, W_ih, W_hh, bih_row, bhh_row, W_out, bout_row)

    return (out_logp, out_h.reshape(1, 1, S))


# speedup vs baseline: 1.0356x; 1.0356x over previous
import jax
import jax.numpy as jnp
from jax.experimental import pallas as pl
from jax.experimental.pallas import tpu as pltpu

I = 1024
S = 1024
O = 1024
G = 8
BR = (3 * S) // G
BO = O // G


def _fused_step(f_ref, h_ref, wih_ref, whh_ref, bih_ref, bhh_ref,
                wout_ref, bout_ref, out_logp_ref, out_h_ref,
                gi_ref, gh_ref, lg_ref):
    g = pl.program_id(0)

    f_row = f_ref[...]
    h_row = h_ref[...]

    gi = jnp.sum(wih_ref[...] * f_row, axis=1, keepdims=True)
    gh = jnp.sum(whh_ref[...] * h_row, axis=1, keepdims=True)
    gi_ref[0, pl.ds(g * BR, BR)] = jnp.transpose(gi)[0]
    gh_ref[0, pl.ds(g * BR, BR)] = jnp.transpose(gh)[0]

    pred = jnp.concatenate([f_row, h_row, h_row], axis=1)
    lo = jnp.sum(wout_ref[:, :3 * S] * pred, axis=1, keepdims=True)
    lg_ref[0, pl.ds(g * BO, BO)] = jnp.transpose(lo)[0]

    @pl.when(g == G - 1)
    def _finish():
        gi_full = gi_ref[...] + bih_ref[...]
        gh_full = gh_ref[...] + bhh_ref[...]
        r = jax.nn.sigmoid(gi_full[:, :S] + gh_full[:, :S])
        z = jax.nn.sigmoid(gi_full[:, S:2 * S] + gh_full[:, S:2 * S])
        n = jnp.tanh(gi_full[:, 2 * S:] + r * gh_full[:, 2 * S:])
        out_h_ref[...] = (1.0 - z) * n + z * h_row

        logits = lg_ref[...] + bout_ref[...]
        m = jnp.max(logits)
        lse = jnp.log(jnp.sum(jnp.exp(logits - m))) + m
        out_logp_ref[...] = logits - lse


def kernel(feature, time, initial_h, W_ih, W_hh, b_ih, b_hh, W_out, b_out):
    del time
    f_row = feature.reshape(1, I)
    h_row = initial_h.reshape(1, S)
    bih_row = b_ih.reshape(1, 3 * S)
    bhh_row = b_hh.reshape(1, 3 * S)
    bout_row = b_out.reshape(1, O)

    out_logp, out_h = pl.pallas_call(
        _fused_step,
        grid=(G,),
        in_specs=[
            pl.BlockSpec((1, I), lambda g: (0, 0)),
            pl.BlockSpec((1, S), lambda g: (0, 0)),
            pl.BlockSpec((BR, I), lambda g: (g, 0)),
            pl.BlockSpec((BR, S), lambda g: (g, 0)),
            pl.BlockSpec((1, 3 * S), lambda g: (0, 0)),
            pl.BlockSpec((1, 3 * S), lambda g: (0, 0)),
            pl.BlockSpec((BO, W_out.shape[1]), lambda g: (g, 0)),
            pl.BlockSpec((1, O), lambda g: (0, 0)),
        ],
        out_specs=[
            pl.BlockSpec((1, O), lambda g: (0, 0)),
            pl.BlockSpec((1, S), lambda g: (0, 0)),
        ],
        out_shape=[
            jax.ShapeDtypeStruct((1, O), jnp.float32),
            jax.ShapeDtypeStruct((1, S), jnp.float32),
        ],
        scratch_shapes=[
            pltpu.VMEM((1, 3 * S), jnp.float32),
            pltpu.VMEM((1, 3 * S), jnp.float32),
            pltpu.VMEM((1, O), jnp.float32),
        ],
    )(f_row, h_row, W_ih, W_hh, bih_row, bhh_row, W_out, bout_row)

    return (out_logp, out_h.reshape(1, 1, S))


# speedup vs baseline: 1.0671x; 1.0304x over previous
import jax
import jax.numpy as jnp
from jax.experimental import pallas as pl
from jax.experimental.pallas import tpu as pltpu

I = 1024
S = 1024
O = 1024
C = 4
RG = (3 * S) // C
RO = O // C


def _fused(f_ref, h_ref, wih_hbm, whh_hbm, bih_ref, bhh_ref, wout_hbm,
           bout_ref, out_logp_ref, out_h_ref,
           wih_v, whh_v, wout_v, gi_ref, gh_ref, lg_ref, sems):
    for c in range(C):
        pltpu.make_async_copy(
            wih_hbm.at[pl.ds(c * RG, RG), :], wih_v.at[pl.ds(c * RG, RG), :],
            sems.at[3 * c]).start()
        pltpu.make_async_copy(
            whh_hbm.at[pl.ds(c * RG, RG), :], whh_v.at[pl.ds(c * RG, RG), :],
            sems.at[3 * c + 1]).start()
        pltpu.make_async_copy(
            wout_hbm.at[pl.ds(c * RO, RO), :], wout_v.at[pl.ds(c * RO, RO), :],
            sems.at[3 * c + 2]).start()

    f_row = f_ref[...]
    h_row = h_ref[...]
    pred = jnp.concatenate([f_row, h_row, h_row], axis=1)

    for c in range(C):
        pltpu.make_async_copy(
            wih_hbm.at[pl.ds(c * RG, RG), :], wih_v.at[pl.ds(c * RG, RG), :],
            sems.at[3 * c]).wait()
        gi = jnp.sum(wih_v[pl.ds(c * RG, RG), :] * f_row, axis=1,
                     keepdims=True)
        gi_ref[0, pl.ds(c * RG, RG)] = jnp.transpose(gi)[0]

        pltpu.make_async_copy(
            whh_hbm.at[pl.ds(c * RG, RG), :], whh_v.at[pl.ds(c * RG, RG), :],
            sems.at[3 * c + 1]).wait()
        gh = jnp.sum(whh_v[pl.ds(c * RG, RG), :] * h_row, axis=1,
                     keepdims=True)
        gh_ref[0, pl.ds(c * RG, RG)] = jnp.transpose(gh)[0]

        pltpu.make_async_copy(
            wout_hbm.at[pl.ds(c * RO, RO), :], wout_v.at[pl.ds(c * RO, RO), :],
            sems.at[3 * c + 2]).wait()
        lo = jnp.sum(wout_v[pl.ds(c * RO, RO), :3 * S] * pred, axis=1,
                     keepdims=True)
        lg_ref[0, pl.ds(c * RO, RO)] = jnp.transpose(lo)[0]

    gi_full = gi_ref[...] + bih_ref[...]
    gh_full = gh_ref[...] + bhh_ref[...]
    r = jax.nn.sigmoid(gi_full[:, :S] + gh_full[:, :S])
    z = jax.nn.sigmoid(gi_full[:, S:2 * S] + gh_full[:, S:2 * S])
    n = jnp.tanh(gi_full[:, 2 * S:] + r * gh_full[:, 2 * S:])
    out_h_ref[...] = (1.0 - z) * n + z * h_row

    logits = lg_ref[...] + bout_ref[...]
    m = jnp.max(logits)
    lse = jnp.log(jnp.sum(jnp.exp(logits - m))) + m
    out_logp_ref[...] = logits - lse


def kernel(feature, time, initial_h, W_ih, W_hh, b_ih, b_hh, W_out, b_out):
    del time
    f_row = feature.reshape(1, I)
    h_row = initial_h.reshape(1, S)
    bih_row = b_ih.reshape(1, 3 * S)
    bhh_row = b_hh.reshape(1, 3 * S)
    bout_row = b_out.reshape(1, O)

    vmem = pl.BlockSpec(memory_space=pltpu.MemorySpace.VMEM)
    hbm = pl.BlockSpec(memory_space=pltpu.MemorySpace.HBM)

    out_logp, out_h = pl.pallas_call(
        _fused,
        in_specs=[vmem, vmem, hbm, hbm, vmem, vmem, hbm, vmem],
        out_specs=[vmem, vmem],
        out_shape=[
            jax.ShapeDtypeStruct((1, O), jnp.float32),
            jax.ShapeDtypeStruct((1, S), jnp.float32),
        ],
        scratch_shapes=[
            pltpu.MemorySpace.VMEM((3 * S, I), jnp.float32),
            pltpu.MemorySpace.VMEM((3 * S, S), jnp.float32),
            pltpu.MemorySpace.VMEM((O, W_out.shape[1]), jnp.float32),
            pltpu.MemorySpace.VMEM((1, 3 * S), jnp.float32),
            pltpu.MemorySpace.VMEM((1, 3 * S), jnp.float32),
            pltpu.MemorySpace.VMEM((1, O), jnp.float32),
            pltpu.SemaphoreType.DMA((3 * C,)),
        ],
    )(f_row, h_row, W_ih, W_hh, bih_row, bhh_row, W_out, bout_row)

    return (out_logp, out_h.reshape(1, 1, S))
